# baseline (speedup 1.0000x reference)
import functools

import jax
import jax.numpy as jnp
from jax import lax
from jax.experimental import pallas as pl
from jax.experimental.pallas import tpu as pltpu
from jax.experimental.pallas import tpu_sc as plsc

_NC = 2
_NS = 16
_LANES = 16
_NW = _NC * _NS


def _pos_encoding(max_len, d_model):
    even_i = jnp.arange(0, d_model, 2).astype(jnp.float32)
    denominator = jnp.power(10000.0, even_i / d_model)
    position = jnp.arange(max_len).reshape(max_len, 1).astype(jnp.float32)
    even_pe = jnp.sin(position / denominator)
    odd_pe = jnp.cos(position / denominator)
    return jnp.stack([even_pe, odd_pe], axis=2).reshape(max_len, d_model)


def _build_sc_call(T, L, V, D, per_w, C):
    n_chunks = per_w // C
    n_vecs = C * (D // _LANES)
    mesh = plsc.VectorSubcoreMesh(core_axis_name="c", subcore_axis_name="s")

    @functools.partial(
        pl.kernel,
        mesh=mesh,
        out_type=jax.ShapeDtypeStruct((T, D), jnp.float32),
        scratch_types=[
            pltpu.VMEM((per_w,), jnp.int32),
            pltpu.VMEM((C, D), jnp.float32),
            pltpu.VMEM((C, D), jnp.float32),
            pltpu.SemaphoreType.DMA,
        ],
    )
    def emb_kernel(tok_hbm, table_hbm, pe_hbm, out_hbm, idx_v, rows_v, pe_v, sem):
        w = lax.axis_index("s") * _NC + lax.axis_index("c")
        t0 = w * per_w
        l0 = lax.rem(t0, L)
        pltpu.sync_copy(tok_hbm.at[pl.ds(t0, per_w)], idx_v)

        def chunk_body(j, carry):
            cb = j * C
            gat = pltpu.async_copy(
                table_hbm.at[idx_v.at[pl.ds(cb, C)]], rows_v, sem
            )
            pltpu.sync_copy(pe_hbm.at[pl.ds(l0 + cb, C)], pe_v)
            gat.wait()

            def vec_body(i, carry2):
                t = i // (D // _LANES)
                o = lax.rem(i, D // _LANES) * _LANES
                rows_v[t, pl.ds(o, _LANES)] = (
                    rows_v[t, pl.ds(o, _LANES)] + pe_v[t, pl.ds(o, _LANES)]
                )
                return carry2

            lax.fori_loop(0, n_vecs, vec_body, 0)
            pltpu.sync_copy(rows_v, out_hbm.at[pl.ds(t0 + cb, C)])
            return carry

        lax.fori_loop(0, n_chunks, chunk_body, 0)

    return emb_kernel


def kernel(tokens, table):
    B, L = tokens.shape
    V, D = table.shape
    T = B * L
    per_w = T // _NW
    C = 32
    pe = _pos_encoding(L, D)
    emb_kernel = _build_sc_call(T, L, V, D, per_w, C)
    out = emb_kernel(tokens.reshape(T), table, pe)
    return out.reshape(B, L, D)


# speedup vs baseline: 3.0312x; 3.0312x over previous
import functools

import jax
import jax.numpy as jnp
from jax import lax
from jax.experimental import pallas as pl
from jax.experimental.pallas import tpu as pltpu
from jax.experimental.pallas import tpu_sc as plsc

_NC = 2
_NS = 16
_LANES = 16
_NW = _NC * _NS
_NBUF = 3


def _pos_encoding(max_len, d_model):
    even_i = jnp.arange(0, d_model, 2).astype(jnp.float32)
    denominator = jnp.power(10000.0, even_i / d_model)
    position = jnp.arange(max_len).reshape(max_len, 1).astype(jnp.float32)
    even_pe = jnp.sin(position / denominator)
    odd_pe = jnp.cos(position / denominator)
    return jnp.stack([even_pe, odd_pe], axis=2).reshape(max_len, d_model)


def _build_sc_call(T, L, B, V, D, per_w, C):
    n_chunks = per_w // C
    ppc = C // B
    pos_per_w = per_w // B
    n_vec = D // _LANES
    mesh = plsc.VectorSubcoreMesh(core_axis_name="c", subcore_axis_name="s")

    scratch = [
        pltpu.VMEM((per_w,), jnp.int32),
        pltpu.VMEM((n_chunks, C), jnp.int32),
    ]
    scratch += [pltpu.VMEM((C, D), jnp.float32) for _ in range(_NBUF)]
    scratch += [pltpu.VMEM((ppc, D), jnp.float32) for _ in range(_NBUF)]
    scratch += [pltpu.SemaphoreType.DMA for _ in range(3 * _NBUF)]

    @functools.partial(
        pl.kernel,
        mesh=mesh,
        out_type=jax.ShapeDtypeStruct((T, D), jnp.float32),
        scratch_types=scratch,
    )
    def emb_kernel(tok_hbm, table_hbm, pe_hbm, oidx_hbm, out_hbm, *sc):
        idx_v, oidx_v = sc[0], sc[1]
        rows = sc[2:2 + _NBUF]
        peb = sc[2 + _NBUF:2 + 2 * _NBUF]
        gsem = sc[2 + 2 * _NBUF:2 + 3 * _NBUF]
        psem = sc[2 + 3 * _NBUF:2 + 4 * _NBUF]
        ssem = sc[2 + 4 * _NBUF:2 + 5 * _NBUF]

        w = lax.axis_index("s") * _NC + lax.axis_index("c")
        t0 = w * per_w
        pe0 = w * pos_per_w
        pltpu.sync_copy(tok_hbm.at[pl.ds(t0, per_w)], idx_v)
        pltpu.sync_copy(oidx_hbm.at[w], oidx_v)

        gat = [None] * _NBUF
        pes = [None] * _NBUF
        sct = [None] * _NBUF

        def start_chunk(j):
            p = j % _NBUF
            gat[p] = pltpu.async_copy(
                table_hbm.at[idx_v.at[pl.ds(j * C, C)]], rows[p], gsem[p]
            )
            pes[p] = pltpu.async_copy(
                pe_hbm.at[pl.ds(pe0 + j * ppc, ppc)], peb[p], psem[p]
            )

        def add_chunk(rows_b, pe_b):
            def body(q, carry):
                o = q * _LANES
                for t in range(ppc):
                    pe_reg = pe_b[t, pl.ds(o, _LANES)]
                    for b in range(B):
                        plsc.addupdate(
                            rows_b.at[t * B + b, pl.ds(o, _LANES)], pe_reg
                        )
                return carry

            lax.fori_loop(0, n_vec, body, 0)

        start_chunk(0)
        start_chunk(1)
        for j in range(n_chunks):
            p = j % _NBUF
            gat[p].wait()
            pes[p].wait()
            add_chunk(rows[p], peb[p])
            sct[p] = pltpu.async_copy(rows[p], out_hbm.at[oidx_v.at[j]], ssem[p])
            nxt = j + _NBUF - 1
            if nxt < n_chunks:
                q = nxt % _NBUF
                if sct[q] is not None:
                    sct[q].wait()
                start_chunk(nxt)
        for p in range(_NBUF):
            sct[p].wait()

    return emb_kernel


def kernel(tokens, table):
    B, L = tokens.shape
    V, D = table.shape
    T = B * L
    per_w = T // _NW
    C = 32
    pe = _pos_encoding(L, D)
    tok_t = tokens.T.reshape(T)
    u = jnp.arange(T, dtype=jnp.int32)
    out_row = (u % B) * L + (u // B)
    oidx = out_row.reshape(_NW, per_w // C, C)
    emb_kernel = _build_sc_call(T, L, B, V, D, per_w, C)
    out = emb_kernel(tok_t, table, pe, oidx)
    return out.reshape(B, L, D)
